# baseline (speedup 1.0000x reference)
import functools

import jax
import jax.numpy as jnp
from jax import lax
from jax.experimental import pallas as pl
from jax.experimental.pallas import tpu as pltpu
from jax.experimental.pallas import tpu_sc as plsc

N = 10000
E = 160000
F = 256

NC = 2
NS = 16
NT = NC * NS

RANGE = 320
NP = NT * RANGE
DUMMY = RANGE
EP = 5120
EPAD = NT * EP
PAD_DST = NP - 1
STRIP = 1024
NCH = EP // 16
ELIST = NT * NT * EP

RB = 1024
DB = 1000


def _route_body(src_hbm, dst_hbm, hist_out, cnt_out, elist_out,
                bufs_v, bufd_v, hist_v, ctr_v, idx_v, val_v,
                ssem0, ssem1, ssem2, ssem3):
    c = lax.axis_index("c")
    s = lax.axis_index("s")
    t = c * NS + s
    ebase = t * EP
    lanes = lax.iota(jnp.int32, 16)
    ssems = (ssem0, ssem1, ssem2, ssem3)

    def zhist(i, carry):
        hist_v[pl.ds(16 * i, 16)] = jnp.zeros((16,), jnp.float32)
        return carry
    lax.fori_loop(0, NP // 16, zhist, 0)

    def zctr(i, carry):
        ctr_v[pl.ds(16 * i, 16)] = jnp.zeros((16,), jnp.int32)
        return carry
    lax.fori_loop(0, NT, zctr, 0)

    def strip(si, carry):
        pltpu.sync_copy(src_hbm.at[pl.ds(ebase + STRIP * si, STRIP)], bufs_v)
        pltpu.sync_copy(dst_hbm.at[pl.ds(ebase + STRIP * si, STRIP)], bufd_v)

        def macro(m, carry2):
            for i in range(4):
                j = 4 * m + i
                d = bufd_v[pl.ds(16 * j, 16)]
                sv = bufs_v[pl.ds(16 * j, 16)]
                bv = (d * 6554) >> 21
                dlv = d - bv * RANGE

                @pl.when(si * (STRIP // 16) + j >= 4)
                def _(i=i):
                    pltpu.make_async_copy(
                        val_v.at[i], elist_out.at[pl.ds(0, 16)],
                        ssems[i]).wait()
                val_v[i, pl.ds(0, 16)] = sv * 512 + dlv

                posv = jnp.zeros((16,), jnp.int32)
                for ll in range(16):
                    b_l = bv[ll]
                    cslot = ctr_v[pl.ds(b_l * 16, 16)]
                    posv = jnp.where(lanes == ll, cslot[0], posv)
                    ctr_v[pl.ds(b_l * 16, 16)] = cslot + 1
                    n_l = d[ll]
                    hbase = (n_l >> 4) << 4
                    hl = n_l & 15
                    hrow = hist_v[pl.ds(hbase, 16)]
                    hist_v[pl.ds(hbase, 16)] = hrow + jnp.where(
                        lanes == hl, 1.0, 0.0)
                idx_v[i, pl.ds(0, 16)] = (bv * NT + t) * EP + posv
                pltpu.async_copy(val_v.at[i], elist_out.at[idx_v.at[i]],
                                 ssems[i])
            return carry2
        lax.fori_loop(0, STRIP // 64, macro, carry)
        return carry
    lax.fori_loop(0, EP // STRIP, strip, 0)
    for i in range(4):
        pltpu.make_async_copy(
            val_v.at[i], elist_out.at[pl.ds(0, 16)], ssems[i]).wait()

    pltpu.sync_copy(hist_v, hist_out.at[t])
    pltpu.sync_copy(ctr_v, cnt_out.at[t])


def _make_route():
    mesh = plsc.VectorSubcoreMesh(
        core_axis_name="c", subcore_axis_name="s",
        num_cores=NC, num_subcores=NS)
    return pl.kernel(
        _route_body,
        out_type=[
            jax.ShapeDtypeStruct((NT, NP), jnp.float32),
            jax.ShapeDtypeStruct((NT, 16 * NT), jnp.int32),
            jax.ShapeDtypeStruct((ELIST,), jnp.int32),
        ],
        mesh=mesh,
        scratch_types=[
            pltpu.VMEM((STRIP,), jnp.int32),
            pltpu.VMEM((STRIP,), jnp.int32),
            pltpu.VMEM((NP,), jnp.float32),
            pltpu.VMEM((16 * NT,), jnp.int32),
            pltpu.VMEM((4, 16), jnp.int32),
            pltpu.VMEM((4, 16), jnp.int32),
            pltpu.SemaphoreType.DMA,
            pltpu.SemaphoreType.DMA,
            pltpu.SemaphoreType.DMA,
            pltpu.SemaphoreType.DMA,
        ],
    )


def _mm_body(x_ref, w_ref, deg_ref, h_ref, s_ref):
    deg = jnp.sum(deg_ref[...], axis=0) + 1.0
    s = lax.rsqrt(deg)
    h = jnp.dot(x_ref[...], w_ref[...], preferred_element_type=jnp.float32)
    h_ref[...] = h * s[:, None]
    s_ref[...] = jnp.broadcast_to(s[:, None], (RB, 8))


_mm_kernel = pl.pallas_call(
    _mm_body,
    grid=(NP // RB,),
    in_specs=[
        pl.BlockSpec((RB, F), lambda i: (i, 0)),
        pl.BlockSpec((F, F), lambda i: (0, 0)),
        pl.BlockSpec((NT, RB), lambda i: (0, i)),
    ],
    out_specs=[
        pl.BlockSpec((RB, F), lambda i: (i, 0)),
        pl.BlockSpec((RB, 8), lambda i: (i, 0)),
    ],
    out_shape=[
        jax.ShapeDtypeStruct((NP, F), jnp.float32),
        jax.ShapeDtypeStruct((NP, 8), jnp.float32),
    ],
)


GC = 16
NBUF = 8


def _agg_body(h_hbm, elist_hbm, cnt_hbm, agg_out,
              cnt_v, pk_v, srcl_v, dll_v, rows0_v, rows1_v, rows2_v,
              rows3_v, rows4_v, rows5_v, rows6_v, rows7_v, acc_v,
              sem0, sem1, sem2, sem3, sem4, sem5, sem6, sem7):
    c = lax.axis_index("c")
    s = lax.axis_index("s")
    t = c * NS + s
    base = t * RANGE
    lanes = lax.iota(jnp.int32, 16)

    pltpu.sync_copy(h_hbm.at[pl.ds(base, RANGE)], acc_v.at[pl.ds(0, RANGE)])
    rows = (rows0_v, rows1_v, rows2_v, rows3_v,
            rows4_v, rows5_v, rows6_v, rows7_v)
    sems = (sem0, sem1, sem2, sem3, sem4, sem5, sem6, sem7)

    def producer(p, carry):
        pltpu.sync_copy(cnt_hbm.at[p, pl.ds(t * 16, 16)], cnt_v)
        cnt = cnt_v[pl.ds(0, 16)][0]
        rb = (t * NT + p) * EP
        nch = (cnt + 127) >> 7

        def build(gi, carry2):
            pltpu.sync_copy(elist_hbm.at[pl.ds(rb + 128 * gi, 128)], pk_v)
            for j in range(8):
                v = pk_v[pl.ds(16 * j, 16)]
                pos = 128 * gi + 16 * j + lanes
                ok = pos < cnt
                sl = pl.ds(128 * gi + 16 * j, 16)
                srcl_v[sl] = jnp.where(ok, v >> 9, 0)
                dll_v[sl] = jnp.where(ok, v & 511, DUMMY)
            return carry2
        lax.fori_loop(0, nch, build, carry)

        nb = (cnt + GC - 1) >> 4

        for i in range(NBUF - 1):
            @pl.when(i < nb)
            def _(i=i):
                pltpu.async_copy(h_hbm.at[srcl_v.at[pl.ds(GC * i, GC)]],
                                 rows[i], sems[i])

        def macro(m, carry2):
            for i in range(NBUF):
                ch = NBUF * m + i

                @pl.when(ch < nb)
                def _(i=i, ch=ch):
                    rbuf = rows[i]
                    pltpu.make_async_copy(
                        h_hbm.at[pl.ds(0, GC)], rbuf, sems[i]).wait()
                    nxt = ch + NBUF - 1

                    @pl.when(nxt < nb)
                    def _():
                        pltpu.async_copy(
                            h_hbm.at[srcl_v.at[pl.ds(GC * nxt, GC)]],
                            rows[(i + NBUF - 1) % NBUF],
                            sems[(i + NBUF - 1) % NBUF])
                    for e16 in range(GC // 16):
                        dlc = dll_v[pl.ds(GC * ch + 16 * e16, 16)]
                        for ll in range(16):
                            dl = dlc[ll]
                            for kk in range(F // 16):
                                fs = pl.ds(16 * kk, 16)
                                plsc.addupdate(acc_v.at[dl, fs],
                                               rbuf[16 * e16 + ll, fs])
            return carry2
        lax.fori_loop(0, (nb + NBUF - 1) >> 3, macro, carry)
        return carry
    lax.fori_loop(0, NT, producer, 0)

    pltpu.sync_copy(acc_v.at[pl.ds(0, RANGE)], agg_out.at[pl.ds(base, RANGE)])


def _make_agg():
    mesh = plsc.VectorSubcoreMesh(
        core_axis_name="c", subcore_axis_name="s",
        num_cores=NC, num_subcores=NS)
    return pl.kernel(
        _agg_body,
        out_type=jax.ShapeDtypeStruct((NP, F), jnp.float32),
        mesh=mesh,
        scratch_types=[
            pltpu.VMEM((16,), jnp.int32),
            pltpu.VMEM((128,), jnp.int32),
            pltpu.VMEM((EP,), jnp.int32),
            pltpu.VMEM((EP,), jnp.int32),
            pltpu.VMEM((GC, F), jnp.float32),
            pltpu.VMEM((GC, F), jnp.float32),
            pltpu.VMEM((GC, F), jnp.float32),
            pltpu.VMEM((GC, F), jnp.float32),
            pltpu.VMEM((GC, F), jnp.float32),
            pltpu.VMEM((GC, F), jnp.float32),
            pltpu.VMEM((GC, F), jnp.float32),
            pltpu.VMEM((GC, F), jnp.float32),
            pltpu.VMEM((RANGE + 8, F), jnp.float32),
            pltpu.SemaphoreType.DMA,
            pltpu.SemaphoreType.DMA,
            pltpu.SemaphoreType.DMA,
            pltpu.SemaphoreType.DMA,
            pltpu.SemaphoreType.DMA,
            pltpu.SemaphoreType.DMA,
            pltpu.SemaphoreType.DMA,
            pltpu.SemaphoreType.DMA,
        ],
    )


def _fin_body(agg_ref, s_ref, b_ref, out_ref):
    s = s_ref[...][:, 0:1]
    out_ref[...] = jnp.maximum(agg_ref[...] * s + b_ref[...], 0.0)


_fin_kernel = pl.pallas_call(
    _fin_body,
    grid=(N // DB,),
    in_specs=[
        pl.BlockSpec((DB, F), lambda i: (i, 0)),
        pl.BlockSpec((DB, 8), lambda i: (i, 0)),
        pl.BlockSpec((1, F), lambda i: (0, 0)),
    ],
    out_specs=pl.BlockSpec((DB, F), lambda i: (i, 0)),
    out_shape=jax.ShapeDtypeStruct((N, F), jnp.float32),
)


@functools.lru_cache(maxsize=None)
def _sc_kernels():
    return _make_route(), _make_agg()


def kernel(x, edge_index, W, b):
    route_k, agg_k = _sc_kernels()
    src = jnp.concatenate(
        [edge_index[0], jnp.zeros((EPAD - E,), jnp.int32)])
    dst = jnp.concatenate(
        [edge_index[1], jnp.full((EPAD - E,), PAD_DST, jnp.int32)])
    hist, counts, elist = route_k(src, dst)
    h_scaled, s_col = _mm_kernel(x, W, hist)
    agg = agg_k(h_scaled, elist, counts)
    out = _fin_kernel(agg, s_col, b.reshape(1, F))
    return out


# speedup vs baseline: 1.0027x; 1.0027x over previous
import functools

import jax
import jax.numpy as jnp
from jax import lax
from jax.experimental import pallas as pl
from jax.experimental.pallas import tpu as pltpu
from jax.experimental.pallas import tpu_sc as plsc

N = 10000
E = 160000
F = 256

NC = 2
NS = 16
NT = NC * NS

RANGE = 320
NP = NT * RANGE
DUMMY = RANGE
EP = 5120
EPAD = NT * EP
PAD_DST = NP - 1
STRIP = 1024
NCH = EP // 16
ELIST = NT * NT * EP

RB = 1024
DB = 1000


def _route_body(src_hbm, dst_hbm, hist_out, cnt_out, elist_out,
                bufs_v, bufd_v, hist_v, ctr_v, idx_v, val_v,
                ssem0, ssem1, ssem2, ssem3):
    c = lax.axis_index("c")
    s = lax.axis_index("s")
    t = c * NS + s
    ebase = t * EP
    lanes = lax.iota(jnp.int32, 16)
    ssems = (ssem0, ssem1, ssem2, ssem3)

    def zhist(i, carry):
        hist_v[pl.ds(16 * i, 16)] = jnp.zeros((16,), jnp.float32)
        return carry
    lax.fori_loop(0, NP // 16, zhist, 0)

    def zctr(i, carry):
        ctr_v[pl.ds(16 * i, 16)] = jnp.zeros((16,), jnp.int32)
        return carry
    lax.fori_loop(0, NT, zctr, 0)

    def strip(si, carry):
        pltpu.sync_copy(src_hbm.at[pl.ds(ebase + STRIP * si, STRIP)], bufs_v)
        pltpu.sync_copy(dst_hbm.at[pl.ds(ebase + STRIP * si, STRIP)], bufd_v)

        def macro(m, carry2):
            for i in range(4):
                j = 4 * m + i
                d = bufd_v[pl.ds(16 * j, 16)]
                sv = bufs_v[pl.ds(16 * j, 16)]
                bv = (d * 6554) >> 21
                dlv = d - bv * RANGE

                @pl.when(si * (STRIP // 16) + j >= 4)
                def _(i=i):
                    pltpu.make_async_copy(
                        val_v.at[i], elist_out.at[pl.ds(0, 16)],
                        ssems[i]).wait()
                val_v[i, pl.ds(0, 16)] = sv * 512 + dlv

                posv = jnp.zeros((16,), jnp.int32)
                for ll in range(16):
                    b_l = bv[ll]
                    cslot = ctr_v[pl.ds(b_l * 16, 16)]
                    posv = jnp.where(lanes == ll, cslot[0], posv)
                    ctr_v[pl.ds(b_l * 16, 16)] = cslot + 1
                    n_l = d[ll]
                    hbase = (n_l >> 4) << 4
                    hl = n_l & 15
                    plsc.addupdate(hist_v.at[pl.ds(hbase, 16)],
                                   jnp.where(lanes == hl, 1.0, 0.0))
                idx_v[i, pl.ds(0, 16)] = (bv * NT + t) * EP + posv
                pltpu.async_copy(val_v.at[i], elist_out.at[idx_v.at[i]],
                                 ssems[i])
            return carry2
        lax.fori_loop(0, STRIP // 64, macro, carry)
        return carry
    lax.fori_loop(0, EP // STRIP, strip, 0)
    for i in range(4):
        pltpu.make_async_copy(
            val_v.at[i], elist_out.at[pl.ds(0, 16)], ssems[i]).wait()

    pltpu.sync_copy(hist_v, hist_out.at[t])
    pltpu.sync_copy(ctr_v, cnt_out.at[t])


def _make_route():
    mesh = plsc.VectorSubcoreMesh(
        core_axis_name="c", subcore_axis_name="s",
        num_cores=NC, num_subcores=NS)
    return pl.kernel(
        _route_body,
        out_type=[
            jax.ShapeDtypeStruct((NT, NP), jnp.float32),
            jax.ShapeDtypeStruct((NT, 16 * NT), jnp.int32),
            jax.ShapeDtypeStruct((ELIST,), jnp.int32),
        ],
        mesh=mesh,
        scratch_types=[
            pltpu.VMEM((STRIP,), jnp.int32),
            pltpu.VMEM((STRIP,), jnp.int32),
            pltpu.VMEM((NP,), jnp.float32),
            pltpu.VMEM((16 * NT,), jnp.int32),
            pltpu.VMEM((4, 16), jnp.int32),
            pltpu.VMEM((4, 16), jnp.int32),
            pltpu.SemaphoreType.DMA,
            pltpu.SemaphoreType.DMA,
            pltpu.SemaphoreType.DMA,
            pltpu.SemaphoreType.DMA,
        ],
    )


def _mm_body(x_ref, w_ref, deg_ref, h_ref, s_ref):
    deg = jnp.sum(deg_ref[...], axis=0) + 1.0
    s = lax.rsqrt(deg)
    h = jnp.dot(x_ref[...], w_ref[...], preferred_element_type=jnp.float32)
    h_ref[...] = h * s[:, None]
    s_ref[...] = jnp.broadcast_to(s[:, None], (RB, 8))


_mm_kernel = pl.pallas_call(
    _mm_body,
    grid=(NP // RB,),
    in_specs=[
        pl.BlockSpec((RB, F), lambda i: (i, 0)),
        pl.BlockSpec((F, F), lambda i: (0, 0)),
        pl.BlockSpec((NT, RB), lambda i: (0, i)),
    ],
    out_specs=[
        pl.BlockSpec((RB, F), lambda i: (i, 0)),
        pl.BlockSpec((RB, 8), lambda i: (i, 0)),
    ],
    out_shape=[
        jax.ShapeDtypeStruct((NP, F), jnp.float32),
        jax.ShapeDtypeStruct((NP, 8), jnp.float32),
    ],
)


GC = 16
NBUF = 8


def _agg_body(h_hbm, elist_hbm, cnt_hbm, agg_out,
              cnt_v, pk_v, srcl_v, dll_v, rows0_v, rows1_v, rows2_v,
              rows3_v, rows4_v, rows5_v, rows6_v, rows7_v, acc_v,
              sem0, sem1, sem2, sem3, sem4, sem5, sem6, sem7):
    c = lax.axis_index("c")
    s = lax.axis_index("s")
    t = c * NS + s
    base = t * RANGE
    lanes = lax.iota(jnp.int32, 16)

    pltpu.sync_copy(h_hbm.at[pl.ds(base, RANGE)], acc_v.at[pl.ds(0, RANGE)])
    rows = (rows0_v, rows1_v, rows2_v, rows3_v,
            rows4_v, rows5_v, rows6_v, rows7_v)
    sems = (sem0, sem1, sem2, sem3, sem4, sem5, sem6, sem7)

    def producer(p, carry):
        pltpu.sync_copy(cnt_hbm.at[p, pl.ds(t * 16, 16)], cnt_v)
        cnt = cnt_v[pl.ds(0, 16)][0]
        rb = (t * NT + p) * EP
        nch = (cnt + 127) >> 7

        def build(gi, carry2):
            pltpu.sync_copy(elist_hbm.at[pl.ds(rb + 128 * gi, 128)], pk_v)
            for j in range(8):
                v = pk_v[pl.ds(16 * j, 16)]
                pos = 128 * gi + 16 * j + lanes
                ok = pos < cnt
                sl = pl.ds(128 * gi + 16 * j, 16)
                srcl_v[sl] = jnp.where(ok, v >> 9, 0)
                dll_v[sl] = jnp.where(ok, v & 511, DUMMY)
            return carry2
        lax.fori_loop(0, nch, build, carry)

        nb = (cnt + GC - 1) >> 4

        for i in range(NBUF - 1):
            @pl.when(i < nb)
            def _(i=i):
                pltpu.async_copy(h_hbm.at[srcl_v.at[pl.ds(GC * i, GC)]],
                                 rows[i], sems[i])

        def macro(m, carry2):
            for i in range(NBUF):
                ch = NBUF * m + i

                @pl.when(ch < nb)
                def _(i=i, ch=ch):
                    rbuf = rows[i]
                    pltpu.make_async_copy(
                        h_hbm.at[pl.ds(0, GC)], rbuf, sems[i]).wait()
                    nxt = ch + NBUF - 1

                    @pl.when(nxt < nb)
                    def _():
                        pltpu.async_copy(
                            h_hbm.at[srcl_v.at[pl.ds(GC * nxt, GC)]],
                            rows[(i + NBUF - 1) % NBUF],
                            sems[(i + NBUF - 1) % NBUF])
                    for e16 in range(GC // 16):
                        dlc = dll_v[pl.ds(GC * ch + 16 * e16, 16)]
                        for ll in range(16):
                            dl = dlc[ll]
                            for kk in range(F // 16):
                                fs = pl.ds(16 * kk, 16)
                                plsc.addupdate(acc_v.at[dl, fs],
                                               rbuf[16 * e16 + ll, fs])
            return carry2
        lax.fori_loop(0, (nb + NBUF - 1) >> 3, macro, carry)
        return carry
    lax.fori_loop(0, NT, producer, 0)

    pltpu.sync_copy(acc_v.at[pl.ds(0, RANGE)], agg_out.at[pl.ds(base, RANGE)])


def _make_agg():
    mesh = plsc.VectorSubcoreMesh(
        core_axis_name="c", subcore_axis_name="s",
        num_cores=NC, num_subcores=NS)
    return pl.kernel(
        _agg_body,
        out_type=jax.ShapeDtypeStruct((NP, F), jnp.float32),
        mesh=mesh,
        scratch_types=[
            pltpu.VMEM((16,), jnp.int32),
            pltpu.VMEM((128,), jnp.int32),
            pltpu.VMEM((EP,), jnp.int32),
            pltpu.VMEM((EP,), jnp.int32),
            pltpu.VMEM((GC, F), jnp.float32),
            pltpu.VMEM((GC, F), jnp.float32),
            pltpu.VMEM((GC, F), jnp.float32),
            pltpu.VMEM((GC, F), jnp.float32),
            pltpu.VMEM((GC, F), jnp.float32),
            pltpu.VMEM((GC, F), jnp.float32),
            pltpu.VMEM((GC, F), jnp.float32),
            pltpu.VMEM((GC, F), jnp.float32),
            pltpu.VMEM((RANGE + 8, F), jnp.float32),
            pltpu.SemaphoreType.DMA,
            pltpu.SemaphoreType.DMA,
            pltpu.SemaphoreType.DMA,
            pltpu.SemaphoreType.DMA,
            pltpu.SemaphoreType.DMA,
            pltpu.SemaphoreType.DMA,
            pltpu.SemaphoreType.DMA,
            pltpu.SemaphoreType.DMA,
        ],
    )


def _fin_body(agg_ref, s_ref, b_ref, out_ref):
    s = s_ref[...][:, 0:1]
    out_ref[...] = jnp.maximum(agg_ref[...] * s + b_ref[...], 0.0)


_fin_kernel = pl.pallas_call(
    _fin_body,
    grid=(N // DB,),
    in_specs=[
        pl.BlockSpec((DB, F), lambda i: (i, 0)),
        pl.BlockSpec((DB, 8), lambda i: (i, 0)),
        pl.BlockSpec((1, F), lambda i: (0, 0)),
    ],
    out_specs=pl.BlockSpec((DB, F), lambda i: (i, 0)),
    out_shape=jax.ShapeDtypeStruct((N, F), jnp.float32),
)


@functools.lru_cache(maxsize=None)
def _sc_kernels():
    return _make_route(), _make_agg()


def kernel(x, edge_index, W, b):
    route_k, agg_k = _sc_kernels()
    src = jnp.concatenate(
        [edge_index[0], jnp.zeros((EPAD - E,), jnp.int32)])
    dst = jnp.concatenate(
        [edge_index[1], jnp.full((EPAD - E,), PAD_DST, jnp.int32)])
    hist, counts, elist = route_k(src, dst)
    h_scaled, s_col = _mm_kernel(x, W, hist)
    agg = agg_k(h_scaled, elist, counts)
    out = _fin_kernel(agg, s_col, b.reshape(1, F))
    return out


# speedup vs baseline: 1.0772x; 1.0742x over previous
import functools

import jax
import jax.numpy as jnp
from jax import lax
from jax.experimental import pallas as pl
from jax.experimental.pallas import tpu as pltpu
from jax.experimental.pallas import tpu_sc as plsc

N = 10000
E = 160000
F = 256

NC = 2
NS = 16
NT = NC * NS

RANGE = 320
NP = NT * RANGE
DUMMY = RANGE
EP = 5120
EPAD = NT * EP
PAD_DST = NP - 1
STRIP = 1024
NCH = EP // 16
ELIST = NT * NT * EP

RB = 1024
DB = 1000


def _route_body(src_hbm, dst_hbm, hist_out, cnt_out, elist_out,
                bufs_v, bufd_v, hist_v, ctr_v, idx_v, val_v,
                ssem0, ssem1, ssem2, ssem3):
    c = lax.axis_index("c")
    s = lax.axis_index("s")
    t = c * NS + s
    ebase = t * EP
    lanes = lax.iota(jnp.int32, 16)
    ssems = (ssem0, ssem1, ssem2, ssem3)

    def zhist(i, carry):
        hist_v[pl.ds(16 * i, 16)] = jnp.zeros((16,), jnp.float32)
        return carry
    lax.fori_loop(0, NP // 16, zhist, 0)

    def zctr(i, carry):
        ctr_v[pl.ds(16 * i, 16)] = jnp.zeros((16,), jnp.int32)
        return carry
    lax.fori_loop(0, NT, zctr, 0)

    def strip(si, carry):
        pltpu.sync_copy(src_hbm.at[pl.ds(ebase + STRIP * si, STRIP)], bufs_v)
        pltpu.sync_copy(dst_hbm.at[pl.ds(ebase + STRIP * si, STRIP)], bufd_v)

        def macro(m, carry2):
            for i in range(4):
                j = 4 * m + i
                d = bufd_v[pl.ds(16 * j, 16)]
                sv = bufs_v[pl.ds(16 * j, 16)]
                bv = (d * 6554) >> 21
                dlv = d - bv * RANGE

                @pl.when(si * (STRIP // 16) + j >= 4)
                def _(i=i):
                    pltpu.make_async_copy(
                        val_v.at[i], elist_out.at[pl.ds(0, 16)],
                        ssems[i]).wait()
                val_v[i, pl.ds(0, 16)] = sv * 512 + dlv

                posv = jnp.zeros((16,), jnp.int32)
                for ll in range(16):
                    b_l = bv[ll]
                    cslot = ctr_v[pl.ds(b_l * 16, 16)]
                    posv = jnp.where(lanes == ll, cslot[0], posv)
                    ctr_v[pl.ds(b_l * 16, 16)] = cslot + 1
                    n_l = d[ll]
                    hbase = (n_l >> 4) << 4
                    hl = n_l & 15
                    plsc.addupdate(hist_v.at[pl.ds(hbase, 16)],
                                   jnp.where(lanes == hl, 1.0, 0.0))
                idx_v[i, pl.ds(0, 16)] = (bv * NT + t) * EP + posv
                pltpu.async_copy(val_v.at[i], elist_out.at[idx_v.at[i]],
                                 ssems[i])
            return carry2
        lax.fori_loop(0, STRIP // 64, macro, carry)
        return carry
    lax.fori_loop(0, EP // STRIP, strip, 0)
    for i in range(4):
        pltpu.make_async_copy(
            val_v.at[i], elist_out.at[pl.ds(0, 16)], ssems[i]).wait()

    pltpu.sync_copy(hist_v, hist_out.at[t])
    pltpu.sync_copy(ctr_v, cnt_out.at[t])


def _make_route():
    mesh = plsc.VectorSubcoreMesh(
        core_axis_name="c", subcore_axis_name="s",
        num_cores=NC, num_subcores=NS)
    return pl.kernel(
        _route_body,
        out_type=[
            jax.ShapeDtypeStruct((NT, NP), jnp.float32),
            jax.ShapeDtypeStruct((NT, 16 * NT), jnp.int32),
            jax.ShapeDtypeStruct((ELIST,), jnp.int32),
        ],
        mesh=mesh,
        scratch_types=[
            pltpu.VMEM((STRIP,), jnp.int32),
            pltpu.VMEM((STRIP,), jnp.int32),
            pltpu.VMEM((NP,), jnp.float32),
            pltpu.VMEM((16 * NT,), jnp.int32),
            pltpu.VMEM((4, 16), jnp.int32),
            pltpu.VMEM((4, 16), jnp.int32),
            pltpu.SemaphoreType.DMA,
            pltpu.SemaphoreType.DMA,
            pltpu.SemaphoreType.DMA,
            pltpu.SemaphoreType.DMA,
        ],
    )


def _mm_body(x_ref, w_ref, deg_ref, h_ref, hb_ref, s_ref):
    deg = jnp.sum(deg_ref[...], axis=0) + 1.0
    s = lax.rsqrt(deg)
    h = jnp.dot(x_ref[...], w_ref[...], preferred_element_type=jnp.float32)
    hs = h * s[:, None]
    h_ref[...] = hs
    u = lax.bitcast_convert_type(hs, jnp.uint32)
    r = u + jnp.uint32(0x7FFF) + ((u >> 16) & jnp.uint32(1))
    p = (r[:, :F // 2] >> 16) | (r[:, F // 2:] & jnp.uint32(0xFFFF0000))
    hb_ref[...] = lax.bitcast_convert_type(p, jnp.int32)
    s_ref[...] = jnp.broadcast_to(s[:, None], (RB, 8))


_mm_kernel = pl.pallas_call(
    _mm_body,
    grid=(NP // RB,),
    in_specs=[
        pl.BlockSpec((RB, F), lambda i: (i, 0)),
        pl.BlockSpec((F, F), lambda i: (0, 0)),
        pl.BlockSpec((NT, RB), lambda i: (0, i)),
    ],
    out_specs=[
        pl.BlockSpec((RB, F), lambda i: (i, 0)),
        pl.BlockSpec((RB, F // 2), lambda i: (i, 0)),
        pl.BlockSpec((RB, 8), lambda i: (i, 0)),
    ],
    out_shape=[
        jax.ShapeDtypeStruct((NP, F), jnp.float32),
        jax.ShapeDtypeStruct((NP, F // 2), jnp.int32),
        jax.ShapeDtypeStruct((NP, 8), jnp.float32),
    ],
)


GC = 16
NBUF = 8


def _agg_body(h_hbm, hb_hbm, elist_hbm, cnt_hbm, agg_out,
              cnt_v, pk_v, srcl_v, dll_v, rows0_v, rows1_v, rows2_v,
              rows3_v, rows4_v, rows5_v, rows6_v, rows7_v, acc_v,
              sem0, sem1, sem2, sem3, sem4, sem5, sem6, sem7):
    c = lax.axis_index("c")
    s = lax.axis_index("s")
    t = c * NS + s
    base = t * RANGE
    lanes = lax.iota(jnp.int32, 16)

    pltpu.sync_copy(h_hbm.at[pl.ds(base, RANGE)], acc_v.at[pl.ds(0, RANGE)])
    rows = (rows0_v, rows1_v, rows2_v, rows3_v,
            rows4_v, rows5_v, rows6_v, rows7_v)
    sems = (sem0, sem1, sem2, sem3, sem4, sem5, sem6, sem7)

    def producer(p, carry):
        pltpu.sync_copy(cnt_hbm.at[p, pl.ds(t * 16, 16)], cnt_v)
        cnt = cnt_v[pl.ds(0, 16)][0]
        rb = (t * NT + p) * EP
        nch = (cnt + 127) >> 7

        def build(gi, carry2):
            pltpu.sync_copy(elist_hbm.at[pl.ds(rb + 128 * gi, 128)], pk_v)
            for j in range(8):
                v = pk_v[pl.ds(16 * j, 16)]
                pos = 128 * gi + 16 * j + lanes
                ok = pos < cnt
                sl = pl.ds(128 * gi + 16 * j, 16)
                srcl_v[sl] = jnp.where(ok, v >> 9, 0)
                dll_v[sl] = jnp.where(ok, v & 511, DUMMY)
            return carry2
        lax.fori_loop(0, nch, build, carry)

        nb = (cnt + GC - 1) >> 4

        for i in range(NBUF - 1):
            @pl.when(i < nb)
            def _(i=i):
                pltpu.async_copy(hb_hbm.at[srcl_v.at[pl.ds(GC * i, GC)]],
                                 rows[i], sems[i])

        def macro(m, carry2):
            for i in range(NBUF):
                ch = NBUF * m + i

                @pl.when(ch < nb)
                def _(i=i, ch=ch):
                    rbuf = rows[i]
                    pltpu.make_async_copy(
                        hb_hbm.at[pl.ds(0, GC)], rbuf, sems[i]).wait()
                    nxt = ch + NBUF - 1

                    @pl.when(nxt < nb)
                    def _():
                        pltpu.async_copy(
                            hb_hbm.at[srcl_v.at[pl.ds(GC * nxt, GC)]],
                            rows[(i + NBUF - 1) % NBUF],
                            sems[(i + NBUF - 1) % NBUF])
                    for e16 in range(GC // 16):
                        dlc = dll_v[pl.ds(GC * ch + 16 * e16, 16)]
                        for ll in range(16):
                            dl = dlc[ll]
                            for kk in range(F // 32):
                                v = rbuf[16 * e16 + ll, pl.ds(16 * kk, 16)]
                                flo = lax.bitcast_convert_type(
                                    v << 16, jnp.float32)
                                fhi = lax.bitcast_convert_type(
                                    v & jnp.int32(-65536), jnp.float32)
                                plsc.addupdate(
                                    acc_v.at[dl, pl.ds(16 * kk, 16)], flo)
                                plsc.addupdate(
                                    acc_v.at[dl, pl.ds(F // 2 + 16 * kk, 16)],
                                    fhi)
            return carry2
        lax.fori_loop(0, (nb + NBUF - 1) >> 3, macro, carry)
        return carry
    lax.fori_loop(0, NT, producer, 0)

    pltpu.sync_copy(acc_v.at[pl.ds(0, RANGE)], agg_out.at[pl.ds(base, RANGE)])


def _make_agg():
    mesh = plsc.VectorSubcoreMesh(
        core_axis_name="c", subcore_axis_name="s",
        num_cores=NC, num_subcores=NS)
    return pl.kernel(
        _agg_body,
        out_type=jax.ShapeDtypeStruct((NP, F), jnp.float32),
        mesh=mesh,
        scratch_types=[
            pltpu.VMEM((16,), jnp.int32),
            pltpu.VMEM((128,), jnp.int32),
            pltpu.VMEM((EP,), jnp.int32),
            pltpu.VMEM((EP,), jnp.int32),
            pltpu.VMEM((GC, F // 2), jnp.int32),
            pltpu.VMEM((GC, F // 2), jnp.int32),
            pltpu.VMEM((GC, F // 2), jnp.int32),
            pltpu.VMEM((GC, F // 2), jnp.int32),
            pltpu.VMEM((GC, F // 2), jnp.int32),
            pltpu.VMEM((GC, F // 2), jnp.int32),
            pltpu.VMEM((GC, F // 2), jnp.int32),
            pltpu.VMEM((GC, F // 2), jnp.int32),
            pltpu.VMEM((RANGE + 8, F), jnp.float32),
            pltpu.SemaphoreType.DMA,
            pltpu.SemaphoreType.DMA,
            pltpu.SemaphoreType.DMA,
            pltpu.SemaphoreType.DMA,
            pltpu.SemaphoreType.DMA,
            pltpu.SemaphoreType.DMA,
            pltpu.SemaphoreType.DMA,
            pltpu.SemaphoreType.DMA,
        ],
    )


def _fin_body(agg_ref, s_ref, b_ref, out_ref):
    s = s_ref[...][:, 0:1]
    out_ref[...] = jnp.maximum(agg_ref[...] * s + b_ref[...], 0.0)


_fin_kernel = pl.pallas_call(
    _fin_body,
    grid=(N // DB,),
    in_specs=[
        pl.BlockSpec((DB, F), lambda i: (i, 0)),
        pl.BlockSpec((DB, 8), lambda i: (i, 0)),
        pl.BlockSpec((1, F), lambda i: (0, 0)),
    ],
    out_specs=pl.BlockSpec((DB, F), lambda i: (i, 0)),
    out_shape=jax.ShapeDtypeStruct((N, F), jnp.float32),
)


@functools.lru_cache(maxsize=None)
def _sc_kernels():
    return _make_route(), _make_agg()


def kernel(x, edge_index, W, b):
    route_k, agg_k = _sc_kernels()
    src = jnp.concatenate(
        [edge_index[0], jnp.zeros((EPAD - E,), jnp.int32)])
    dst = jnp.concatenate(
        [edge_index[1], jnp.full((EPAD - E,), PAD_DST, jnp.int32)])
    hist, counts, elist = route_k(src, dst)
    h_scaled, h_bf16, s_col = _mm_kernel(x, W, hist)
    agg = agg_k(h_scaled, h_bf16, elist, counts)
    out = _fin_kernel(agg, s_col, b.reshape(1, F))
    return out


# speedup vs baseline: 1.0788x; 1.0015x over previous
import functools

import jax
import jax.numpy as jnp
from jax import lax
from jax.experimental import pallas as pl
from jax.experimental.pallas import tpu as pltpu
from jax.experimental.pallas import tpu_sc as plsc

N = 10000
E = 160000
F = 256

NC = 2
NS = 16
NT = NC * NS

RANGE = 320
NP = NT * RANGE
DUMMY = RANGE
EP = 5120
EPAD = NT * EP
PAD_DST = NP - 1
STRIP = 1024
NCH = EP // 16
ELIST = NT * NT * EP

RB = 1024
DB = 1000


def _route_body(src_hbm, dst_hbm, hist_out, cnt_out, elist_out,
                bufs_v, bufd_v, hist_v, ctr_v, idx_v, val_v,
                ssem0, ssem1, ssem2, ssem3):
    c = lax.axis_index("c")
    s = lax.axis_index("s")
    t = c * NS + s
    ebase = t * EP
    lanes = lax.iota(jnp.int32, 16)
    ssems = (ssem0, ssem1, ssem2, ssem3)

    def zhist(i, carry):
        hist_v[pl.ds(16 * i, 16)] = jnp.zeros((16,), jnp.float32)
        return carry
    lax.fori_loop(0, NP // 16, zhist, 0)

    def zctr(i, carry):
        ctr_v[pl.ds(16 * i, 16)] = jnp.zeros((16,), jnp.int32)
        return carry
    lax.fori_loop(0, NT, zctr, 0)

    def strip(si, carry):
        pltpu.sync_copy(src_hbm.at[pl.ds(ebase + STRIP * si, STRIP)], bufs_v)
        pltpu.sync_copy(dst_hbm.at[pl.ds(ebase + STRIP * si, STRIP)], bufd_v)

        def macro(m, carry2):
            for i in range(4):
                j = 4 * m + i
                d = bufd_v[pl.ds(16 * j, 16)]
                sv = bufs_v[pl.ds(16 * j, 16)]
                bv = (d * 6554) >> 21
                dlv = d - bv * RANGE

                @pl.when(si * (STRIP // 16) + j >= 4)
                def _(i=i):
                    pltpu.make_async_copy(
                        val_v.at[i], elist_out.at[pl.ds(0, 16)],
                        ssems[i]).wait()
                val_v[i, pl.ds(0, 16)] = sv * 512 + dlv

                posv = jnp.zeros((16,), jnp.int32)
                for ll in range(16):
                    b_l = bv[ll]
                    cslot = ctr_v[pl.ds(b_l * 16, 16)]
                    posv = jnp.where(lanes == ll, cslot[0], posv)
                    ctr_v[pl.ds(b_l * 16, 16)] = cslot + 1
                    n_l = d[ll]
                    hbase = (n_l >> 4) << 4
                    hl = n_l & 15
                    plsc.addupdate(hist_v.at[pl.ds(hbase, 16)],
                                   jnp.where(lanes == hl, 1.0, 0.0))
                idx_v[i, pl.ds(0, 16)] = (bv * NT + t) * EP + posv
                pltpu.async_copy(val_v.at[i], elist_out.at[idx_v.at[i]],
                                 ssems[i])
            return carry2
        lax.fori_loop(0, STRIP // 64, macro, carry)
        return carry
    lax.fori_loop(0, EP // STRIP, strip, 0)
    for i in range(4):
        pltpu.make_async_copy(
            val_v.at[i], elist_out.at[pl.ds(0, 16)], ssems[i]).wait()

    pltpu.sync_copy(hist_v, hist_out.at[t])
    pltpu.sync_copy(ctr_v, cnt_out.at[t])


def _make_route():
    mesh = plsc.VectorSubcoreMesh(
        core_axis_name="c", subcore_axis_name="s",
        num_cores=NC, num_subcores=NS)
    return pl.kernel(
        _route_body,
        out_type=[
            jax.ShapeDtypeStruct((NT, NP), jnp.float32),
            jax.ShapeDtypeStruct((NT, 16 * NT), jnp.int32),
            jax.ShapeDtypeStruct((ELIST,), jnp.int32),
        ],
        mesh=mesh,
        scratch_types=[
            pltpu.VMEM((STRIP,), jnp.int32),
            pltpu.VMEM((STRIP,), jnp.int32),
            pltpu.VMEM((NP,), jnp.float32),
            pltpu.VMEM((16 * NT,), jnp.int32),
            pltpu.VMEM((4, 16), jnp.int32),
            pltpu.VMEM((4, 16), jnp.int32),
            pltpu.SemaphoreType.DMA,
            pltpu.SemaphoreType.DMA,
            pltpu.SemaphoreType.DMA,
            pltpu.SemaphoreType.DMA,
        ],
    )


def _mm_body(x_ref, w_ref, deg_ref, h_ref, hb_ref, s_ref):
    deg = jnp.sum(deg_ref[...], axis=0) + 1.0
    s = lax.rsqrt(deg)
    h = jnp.dot(x_ref[...], w_ref[...], preferred_element_type=jnp.float32)
    hs = h * s[:, None]
    h_ref[...] = hs
    u = lax.bitcast_convert_type(hs, jnp.uint32)
    r = u + jnp.uint32(0x7FFF) + ((u >> 16) & jnp.uint32(1))
    p = (r[:, :F // 2] >> 16) | (r[:, F // 2:] & jnp.uint32(0xFFFF0000))
    hb_ref[...] = lax.bitcast_convert_type(p, jnp.int32)
    s_ref[...] = jnp.broadcast_to(s[:, None], (RB, 8))


_mm_kernel = pl.pallas_call(
    _mm_body,
    grid=(NP // RB,),
    in_specs=[
        pl.BlockSpec((RB, F), lambda i: (i, 0)),
        pl.BlockSpec((F, F), lambda i: (0, 0)),
        pl.BlockSpec((NT, RB), lambda i: (0, i)),
    ],
    out_specs=[
        pl.BlockSpec((RB, F), lambda i: (i, 0)),
        pl.BlockSpec((RB, F // 2), lambda i: (i, 0)),
        pl.BlockSpec((RB, 8), lambda i: (i, 0)),
    ],
    out_shape=[
        jax.ShapeDtypeStruct((NP, F), jnp.float32),
        jax.ShapeDtypeStruct((NP, F // 2), jnp.int32),
        jax.ShapeDtypeStruct((NP, 8), jnp.float32),
    ],
)


GC = 16
GSH = 4
NBUF = 8


def _agg_body(h_hbm, hb_hbm, elist_hbm, cnt_hbm, agg_out,
              cnt_v, pk_v, srcl_v, dll_v, rows0_v, rows1_v, rows2_v,
              rows3_v, rows4_v, rows5_v, rows6_v, rows7_v, acc_v,
              sem0, sem1, sem2, sem3, sem4, sem5, sem6, sem7):
    c = lax.axis_index("c")
    s = lax.axis_index("s")
    t = c * NS + s
    base = t * RANGE
    lanes = lax.iota(jnp.int32, 16)

    pltpu.sync_copy(h_hbm.at[pl.ds(base, RANGE)], acc_v.at[pl.ds(0, RANGE)])
    rows = (rows0_v, rows1_v, rows2_v, rows3_v,
            rows4_v, rows5_v, rows6_v, rows7_v)
    sems = (sem0, sem1, sem2, sem3, sem4, sem5, sem6, sem7)

    def producer(p, carry):
        pltpu.sync_copy(cnt_hbm.at[p, pl.ds(t * 16, 16)], cnt_v)
        cnt = cnt_v[pl.ds(0, 16)][0]
        rb = (t * NT + p) * EP
        nch = (cnt + 127) >> 7

        def build(gi, carry2):
            pltpu.sync_copy(elist_hbm.at[pl.ds(rb + 128 * gi, 128)], pk_v)
            for j in range(8):
                v = pk_v[pl.ds(16 * j, 16)]
                pos = 128 * gi + 16 * j + lanes
                ok = pos < cnt
                sl = pl.ds(128 * gi + 16 * j, 16)
                srcl_v[sl] = jnp.where(ok, v >> 9, 0)
                dll_v[sl] = jnp.where(ok, v & 511, DUMMY)
            return carry2
        lax.fori_loop(0, nch, build, carry)

        nb = (cnt + GC - 1) >> GSH

        for i in range(NBUF - 1):
            @pl.when(i < nb)
            def _(i=i):
                pltpu.async_copy(hb_hbm.at[srcl_v.at[pl.ds(GC * i, GC)]],
                                 rows[i], sems[i])

        def macro(m, carry2):
            for i in range(NBUF):
                ch = NBUF * m + i

                @pl.when(ch < nb)
                def _(i=i, ch=ch):
                    rbuf = rows[i]
                    pltpu.make_async_copy(
                        hb_hbm.at[pl.ds(0, GC)], rbuf, sems[i]).wait()
                    nxt = ch + NBUF - 1

                    @pl.when(nxt < nb)
                    def _():
                        pltpu.async_copy(
                            hb_hbm.at[srcl_v.at[pl.ds(GC * nxt, GC)]],
                            rows[(i + NBUF - 1) % NBUF],
                            sems[(i + NBUF - 1) % NBUF])
                    for e16 in range(GC // 16):
                        dlc = dll_v[pl.ds(GC * ch + 16 * e16, 16)]
                        for ll in range(16):
                            dl = dlc[ll]
                            for kk in range(F // 32):
                                v = rbuf[16 * e16 + ll, pl.ds(16 * kk, 16)]
                                flo = lax.bitcast_convert_type(
                                    v << 16, jnp.float32)
                                fhi = lax.bitcast_convert_type(
                                    v & jnp.int32(-65536), jnp.float32)
                                plsc.addupdate(
                                    acc_v.at[dl, pl.ds(16 * kk, 16)], flo)
                                plsc.addupdate(
                                    acc_v.at[dl, pl.ds(F // 2 + 16 * kk, 16)],
                                    fhi)
            return carry2
        lax.fori_loop(0, (nb + NBUF - 1) >> 3, macro, carry)
        return carry
    lax.fori_loop(0, NT, producer, 0)

    pltpu.sync_copy(acc_v.at[pl.ds(0, RANGE)], agg_out.at[pl.ds(base, RANGE)])


def _make_agg():
    mesh = plsc.VectorSubcoreMesh(
        core_axis_name="c", subcore_axis_name="s",
        num_cores=NC, num_subcores=NS)
    return pl.kernel(
        _agg_body,
        out_type=jax.ShapeDtypeStruct((NP, F), jnp.float32),
        mesh=mesh,
        scratch_types=[
            pltpu.VMEM((16,), jnp.int32),
            pltpu.VMEM((128,), jnp.int32),
            pltpu.VMEM((EP,), jnp.int32),
            pltpu.VMEM((EP,), jnp.int32),
            pltpu.VMEM((GC, F // 2), jnp.int32),
            pltpu.VMEM((GC, F // 2), jnp.int32),
            pltpu.VMEM((GC, F // 2), jnp.int32),
            pltpu.VMEM((GC, F // 2), jnp.int32),
            pltpu.VMEM((GC, F // 2), jnp.int32),
            pltpu.VMEM((GC, F // 2), jnp.int32),
            pltpu.VMEM((GC, F // 2), jnp.int32),
            pltpu.VMEM((GC, F // 2), jnp.int32),
            pltpu.VMEM((RANGE + 8, F), jnp.float32),
            pltpu.SemaphoreType.DMA,
            pltpu.SemaphoreType.DMA,
            pltpu.SemaphoreType.DMA,
            pltpu.SemaphoreType.DMA,
            pltpu.SemaphoreType.DMA,
            pltpu.SemaphoreType.DMA,
            pltpu.SemaphoreType.DMA,
            pltpu.SemaphoreType.DMA,
        ],
    )


def _fin_body(agg_ref, s_ref, b_ref, out_ref):
    s = s_ref[...][:, 0:1]
    out_ref[...] = jnp.maximum(agg_ref[...] * s + b_ref[...], 0.0)


_fin_kernel = pl.pallas_call(
    _fin_body,
    grid=(N // DB,),
    in_specs=[
        pl.BlockSpec((DB, F), lambda i: (i, 0)),
        pl.BlockSpec((DB, 8), lambda i: (i, 0)),
        pl.BlockSpec((1, F), lambda i: (0, 0)),
    ],
    out_specs=pl.BlockSpec((DB, F), lambda i: (i, 0)),
    out_shape=jax.ShapeDtypeStruct((N, F), jnp.float32),
)


@functools.lru_cache(maxsize=None)
def _sc_kernels():
    return _make_route(), _make_agg()


def kernel(x, edge_index, W, b):
    route_k, agg_k = _sc_kernels()
    src = jnp.concatenate(
        [edge_index[0], jnp.zeros((EPAD - E,), jnp.int32)])
    dst = jnp.concatenate(
        [edge_index[1], jnp.full((EPAD - E,), PAD_DST, jnp.int32)])
    hist, counts, elist = route_k(src, dst)
    h_scaled, h_bf16, s_col = _mm_kernel(x, W, hist)
    agg = agg_k(h_scaled, h_bf16, elist, counts)
    out = _fin_kernel(agg, s_col, b.reshape(1, F))
    return out
